# baseline (speedup 1.0000x reference)
import jax
import jax.numpy as jnp
from jax import lax
from jax.experimental import pallas as pl
from jax.experimental.pallas import tpu as pltpu
from jax.experimental.pallas import tpu_sc as plsc

TAU = 1.0
TOP_C = 16
TOP_K = 8
NUM_EXPERTS = 64
D_MODEL = 4096
N_TOKENS = 16384

BM = 256
NW = 32
LANES = 16
TPW = N_TOKENS // NW
NEG_INF = float("-inf")


def _matmul_body(x_ref, w_ref, logits_ref, probs_ref):
    logits = lax.dot_general(
        x_ref[...], w_ref[...], (((1,), (1,)), ((), ())),
        preferred_element_type=jnp.float32,
    ) / TAU
    logits_ref[...] = logits
    l64 = logits[:, :NUM_EXPERTS]
    m = jnp.max(l64, axis=1, keepdims=True)
    e = jnp.exp(l64 - m)
    probs_ref[...] = e / (jnp.sum(e, axis=1, keepdims=True) + 1e-12)


def _take16(v, idx):
    return lax.gather(
        v,
        idx[:, None],
        lax.GatherDimensionNumbers(
            offset_dims=(), collapsed_slice_dims=(0,), start_index_map=(0,)
        ),
        slice_sizes=(1,),
        mode=lax.GatherScatterMode.PROMISE_IN_BOUNDS,
    )


def _sc_body(tpw, logits_hbm, tki_hbm, tkw_hbm, tci_hbm,
             logits_v, tki_v, tkw_v, tci_v):
    cid = lax.axis_index("c")
    sid = lax.axis_index("s")
    wid = sid * 2 + cid
    base = wid * tpw
    pltpu.sync_copy(logits_hbm.at[pl.ds(base, tpw)], logits_v)

    iota = lax.broadcasted_iota(jnp.int32, (LANES,), 0)
    mask8 = iota < TOP_K
    nxt_perm = jnp.minimum(iota + 1, LANES - 1)
    shift8 = jnp.where(iota >= TOP_K, iota - TOP_K, 0)
    lane0 = jnp.zeros((LANES,), jnp.int32)
    lane15 = jnp.full((LANES,), LANES - 1, jnp.int32)
    xor1 = iota ^ 1
    xor2 = iota ^ 2
    xor4 = iota ^ 4

    def one_token(t):
        ks = [logits_v[t, pl.ds(j * LANES, LANES)] for j in range(4)]

        sk, si = [], []
        for j in range(4):
            a, b = plsc.sort_key_val(ks[j], iota + j * LANES, descending=True)
            sk.append(a)
            si.append(b)

        def merge(ak, ai, bk, bi):
            rak = lax.rev(ak, (0,))
            rai = lax.rev(ai, (0,))
            c = rak >= bk
            wk = jnp.where(c, rak, bk)
            wi = jnp.where(c, rai, bi)
            lk = jnp.where(c, bk, rak)
            skv = plsc.sort_key_val(wk, wi, descending=True)
            return skv[0], skv[1], lk

        k01, i01, l01 = merge(sk[0], si[0], sk[1], si[1])
        k23, i23, l23 = merge(sk[2], si[2], sk[3], si[3])
        fk, fi, lf = merge(k01, i01, k23, i23)

        k16v = _take16(fk, lane15)
        dropped = jnp.maximum(jnp.maximum(l01, l23), lf)
        flag_vec = ((fk == _take16(fk, nxt_perm)) & (iota < LANES - 1)) | (
            dropped == k16v)
        flag = jnp.any(flag_vec)

        def exact(_):
            kk = list(ks)
            ii = [iota + LANES * j for j in range(4)]
            ack = jnp.full((LANES,), NEG_INF, jnp.float32)
            aci = jnp.zeros((LANES,), jnp.int32)
            for r in range(TOP_C):
                mk, mi = kk[0], ii[0]
                for j in range(1, 4):
                    c = (kk[j] > mk) | ((kk[j] == mk) & (ii[j] < mi))
                    mk = jnp.where(c, kk[j], mk)
                    mi = jnp.where(c, ii[j], mi)
                bk = jnp.max(mk)
                bi = jnp.min(jnp.where(mk == bk, mi, NUM_EXPERTS))
                ack = jnp.where(iota == r, bk, ack)
                aci = jnp.where(iota == r, bi, aci)
                kk = [jnp.where(ij == bi, NEG_INF, kj)
                      for kj, ij in zip(kk, ii)]
            return ack, aci

        fk, fi = lax.cond(flag, exact, lambda _: (fk, fi), 0)

        mx8v = _take16(fk, lane0)
        e8 = jnp.where(mask8, jnp.exp(fk - mx8v), 0.0)
        s = e8 + _take16(e8, xor1)
        s = s + _take16(s, xor2)
        s = s + _take16(s, xor4)
        w = e8 / (s + 1e-12)

        tci_v[pl.ds(t * TOP_C, TOP_C)] = fi
        return w, fi

    @plsc.parallel_loop(0, tpw // 2, 1, unroll=2)
    def pair_body(p):
        wa, ia = one_token(2 * p)
        wb, ib = one_token(2 * p + 1)
        tkw_v[pl.ds(p * LANES, LANES)] = jnp.where(mask8, wa,
                                                   _take16(wb, shift8))
        tki_v[pl.ds(p * LANES, LANES)] = jnp.where(mask8, ia,
                                                   _take16(ib, shift8))

    pltpu.sync_copy(tci_v, tci_hbm.at[pl.ds(base * TOP_C, tpw * TOP_C)])
    pltpu.sync_copy(tkw_v, tkw_hbm.at[pl.ds(base * TOP_K, tpw * TOP_K)])
    pltpu.sync_copy(tki_v, tki_hbm.at[pl.ds(base * TOP_K, tpw * TOP_K)])


CHUNK_SIZES = (12288, 4096)


def _make_sc(nt):
    import functools
    tpw = nt // NW
    mesh = plsc.VectorSubcoreMesh(core_axis_name="c", subcore_axis_name="s")
    return pl.kernel(
        functools.partial(_sc_body, tpw),
        out_type=(
            jax.ShapeDtypeStruct((nt * TOP_K,), jnp.int32),
            jax.ShapeDtypeStruct((nt * TOP_K,), jnp.float32),
            jax.ShapeDtypeStruct((nt * TOP_C,), jnp.int32),
        ),
        mesh=mesh,
        compiler_params=pltpu.CompilerParams(needs_layout_passes=False),
        scratch_types=[
            pltpu.VMEM((tpw, 2 * NUM_EXPERTS), jnp.float32),
            pltpu.VMEM((tpw * TOP_K,), jnp.int32),
            pltpu.VMEM((tpw * TOP_K,), jnp.float32),
            pltpu.VMEM((tpw * TOP_C,), jnp.int32),
        ],
    )


@jax.jit
def kernel(x, W):
    w_pad = jnp.concatenate(
        [W, jnp.zeros((NUM_EXPERTS, D_MODEL), jnp.float32)], axis=0)
    probs_parts, sc_outs = [], []
    start = 0
    for cht in CHUNK_SIZES:
        nblk = cht // BM
        off = start // BM
        logits_c, probs_c = pl.pallas_call(
            _matmul_body,
            grid=(nblk,),
            in_specs=[
                pl.BlockSpec((BM, D_MODEL), lambda i, off=off: (off + i, 0)),
                pl.BlockSpec((2 * NUM_EXPERTS, D_MODEL), lambda i: (0, 0)),
            ],
            out_specs=(
                pl.BlockSpec((BM, 2 * NUM_EXPERTS), lambda i: (i, 0)),
                pl.BlockSpec((BM, NUM_EXPERTS), lambda i: (i, 0)),
            ),
            out_shape=(
                jax.ShapeDtypeStruct((cht, 2 * NUM_EXPERTS), jnp.float32),
                jax.ShapeDtypeStruct((cht, NUM_EXPERTS), jnp.float32),
            ),
        )(x, w_pad)
        probs_parts.append(probs_c)
        sc_outs.append(_make_sc(cht)(logits_c))
        start += cht
    if len(CHUNK_SIZES) == 1:
        probs = probs_parts[0]
        tki, tkw, tci = sc_outs[0]
    else:
        probs = jnp.concatenate(probs_parts, axis=0)
        tki, tkw, tci = (jnp.concatenate(parts) for parts in zip(*sc_outs))
    return (
        tki.reshape(N_TOKENS, TOP_K),
        tkw.reshape(N_TOKENS, TOP_K),
        probs,
        tci.reshape(N_TOKENS, TOP_C),
    )


# speedup vs baseline: 1.2450x; 1.2450x over previous
import jax
import jax.numpy as jnp
from jax import lax
from jax.experimental import pallas as pl
from jax.experimental.pallas import tpu as pltpu
from jax.experimental.pallas import tpu_sc as plsc

TAU = 1.0
TOP_C = 16
TOP_K = 8
NUM_EXPERTS = 64
D_MODEL = 4096
N_TOKENS = 16384

BM = 512
NW = 32
LANES = 16
TPW = N_TOKENS // NW
NEG_INF = float("-inf")


def _matmul_body(x_ref, w_ref, logits_ref, probs_ref):
    logits = lax.dot_general(
        x_ref[...], w_ref[...], (((1,), (1,)), ((), ())),
        preferred_element_type=jnp.float32,
    ) / TAU
    logits_ref[...] = logits
    l64 = logits[:, :NUM_EXPERTS]
    m = jnp.max(l64, axis=1, keepdims=True)
    e = jnp.exp(l64 - m)
    probs_ref[...] = e / (jnp.sum(e, axis=1, keepdims=True) + 1e-12)


def _take16(v, idx):
    return lax.gather(
        v,
        idx[:, None],
        lax.GatherDimensionNumbers(
            offset_dims=(), collapsed_slice_dims=(0,), start_index_map=(0,)
        ),
        slice_sizes=(1,),
        mode=lax.GatherScatterMode.PROMISE_IN_BOUNDS,
    )


def _sc_body(tpw, logits_hbm, tki_hbm, tkw_hbm, tci_hbm,
             logits_v, tki_v, tkw_v, tci_v):
    cid = lax.axis_index("c")
    sid = lax.axis_index("s")
    wid = sid * 2 + cid
    base = wid * tpw
    pltpu.sync_copy(logits_hbm.at[pl.ds(base, tpw)], logits_v)

    iota = lax.broadcasted_iota(jnp.int32, (LANES,), 0)
    mask8 = iota < TOP_K
    nxt_perm = jnp.minimum(iota + 1, LANES - 1)
    shift8 = jnp.where(iota >= TOP_K, iota - TOP_K, 0)
    lane0 = jnp.zeros((LANES,), jnp.int32)
    lane15 = jnp.full((LANES,), LANES - 1, jnp.int32)
    xor1 = iota ^ 1
    xor2 = iota ^ 2
    xor4 = iota ^ 4

    def one_token(t):
        ks = [logits_v[t, pl.ds(j * LANES, LANES)] for j in range(4)]

        sk, si = [], []
        for j in range(4):
            a, b = plsc.sort_key_val(ks[j], iota + j * LANES, descending=True)
            sk.append(a)
            si.append(b)

        def merge(ak, ai, bk, bi):
            rak = lax.rev(ak, (0,))
            rai = lax.rev(ai, (0,))
            c = rak >= bk
            wk = jnp.where(c, rak, bk)
            wi = jnp.where(c, rai, bi)
            lk = jnp.where(c, bk, rak)
            skv = plsc.sort_key_val(wk, wi, descending=True)
            return skv[0], skv[1], lk

        k01, i01, l01 = merge(sk[0], si[0], sk[1], si[1])
        k23, i23, l23 = merge(sk[2], si[2], sk[3], si[3])
        fk, fi, lf = merge(k01, i01, k23, i23)

        k16v = _take16(fk, lane15)
        dropped = jnp.maximum(jnp.maximum(l01, l23), lf)
        flag_vec = ((fk == _take16(fk, nxt_perm)) & (iota < LANES - 1)) | (
            dropped == k16v)
        flag = jnp.any(flag_vec)

        def exact(_):
            kk = list(ks)
            ii = [iota + LANES * j for j in range(4)]
            ack = jnp.full((LANES,), NEG_INF, jnp.float32)
            aci = jnp.zeros((LANES,), jnp.int32)
            for r in range(TOP_C):
                mk, mi = kk[0], ii[0]
                for j in range(1, 4):
                    c = (kk[j] > mk) | ((kk[j] == mk) & (ii[j] < mi))
                    mk = jnp.where(c, kk[j], mk)
                    mi = jnp.where(c, ii[j], mi)
                bk = jnp.max(mk)
                bi = jnp.min(jnp.where(mk == bk, mi, NUM_EXPERTS))
                ack = jnp.where(iota == r, bk, ack)
                aci = jnp.where(iota == r, bi, aci)
                kk = [jnp.where(ij == bi, NEG_INF, kj)
                      for kj, ij in zip(kk, ii)]
            return ack, aci

        fk, fi = lax.cond(flag, exact, lambda _: (fk, fi), 0)

        mx8v = _take16(fk, lane0)
        e8 = jnp.where(mask8, jnp.exp(fk - mx8v), 0.0)
        s = e8 + _take16(e8, xor1)
        s = s + _take16(s, xor2)
        s = s + _take16(s, xor4)
        w = e8 / (s + 1e-12)

        tci_v[pl.ds(t * TOP_C, TOP_C)] = fi
        return w, fi

    @plsc.parallel_loop(0, tpw // 2, 1, unroll=2)
    def pair_body(p):
        wa, ia = one_token(2 * p)
        wb, ib = one_token(2 * p + 1)
        tkw_v[pl.ds(p * LANES, LANES)] = jnp.where(mask8, wa,
                                                   _take16(wb, shift8))
        tki_v[pl.ds(p * LANES, LANES)] = jnp.where(mask8, ia,
                                                   _take16(ib, shift8))

    pltpu.sync_copy(tci_v, tci_hbm.at[pl.ds(base * TOP_C, tpw * TOP_C)])
    pltpu.sync_copy(tkw_v, tkw_hbm.at[pl.ds(base * TOP_K, tpw * TOP_K)])
    pltpu.sync_copy(tki_v, tki_hbm.at[pl.ds(base * TOP_K, tpw * TOP_K)])


CHUNK_SIZES = (8192, 8192)


def _make_sc(nt):
    import functools
    tpw = nt // NW
    mesh = plsc.VectorSubcoreMesh(core_axis_name="c", subcore_axis_name="s")
    return pl.kernel(
        functools.partial(_sc_body, tpw),
        out_type=(
            jax.ShapeDtypeStruct((nt * TOP_K,), jnp.int32),
            jax.ShapeDtypeStruct((nt * TOP_K,), jnp.float32),
            jax.ShapeDtypeStruct((nt * TOP_C,), jnp.int32),
        ),
        mesh=mesh,
        compiler_params=pltpu.CompilerParams(needs_layout_passes=False),
        scratch_types=[
            pltpu.VMEM((tpw, 2 * NUM_EXPERTS), jnp.float32),
            pltpu.VMEM((tpw * TOP_K,), jnp.int32),
            pltpu.VMEM((tpw * TOP_K,), jnp.float32),
            pltpu.VMEM((tpw * TOP_C,), jnp.int32),
        ],
    )


@jax.jit
def kernel(x, W):
    w_pad = jnp.concatenate(
        [W, jnp.zeros((NUM_EXPERTS, D_MODEL), jnp.float32)], axis=0)
    probs_parts, sc_outs = [], []
    start = 0
    for cht in CHUNK_SIZES:
        nblk = cht // BM
        off = start // BM
        logits_c, probs_c = pl.pallas_call(
            _matmul_body,
            grid=(nblk,),
            in_specs=[
                pl.BlockSpec((BM, D_MODEL), lambda i, off=off: (off + i, 0)),
                pl.BlockSpec((2 * NUM_EXPERTS, D_MODEL), lambda i: (0, 0)),
            ],
            out_specs=(
                pl.BlockSpec((BM, 2 * NUM_EXPERTS), lambda i: (i, 0)),
                pl.BlockSpec((BM, NUM_EXPERTS), lambda i: (i, 0)),
            ),
            out_shape=(
                jax.ShapeDtypeStruct((cht, 2 * NUM_EXPERTS), jnp.float32),
                jax.ShapeDtypeStruct((cht, NUM_EXPERTS), jnp.float32),
            ),
        )(x, w_pad)
        probs_parts.append(probs_c)
        sc_outs.append(_make_sc(cht)(logits_c))
        start += cht
    if len(CHUNK_SIZES) == 1:
        probs = probs_parts[0]
        tki, tkw, tci = sc_outs[0]
    else:
        probs = jnp.concatenate(probs_parts, axis=0)
        tki, tkw, tci = (jnp.concatenate(parts) for parts in zip(*sc_outs))
    return (
        tki.reshape(N_TOKENS, TOP_K),
        tkw.reshape(N_TOKENS, TOP_K),
        probs,
        tci.reshape(N_TOKENS, TOP_C),
    )
